# baseline (speedup 1.0000x reference)
import functools

import jax
import jax.numpy as jnp
from jax import lax
from jax.experimental import pallas as pl
from jax.experimental.pallas import tpu as pltpu
from jax.experimental.pallas import tpu_sc as plsc

N = 10000
E = 160000
L = 8192
H = 256
HH = H // 2
NC, NS, LN = 2, 16, 16
CHUNK = 128
NCHUNK = E // CHUNK
ROWS_PER_SUB = N // NS
ZROWS = 25
BM = 1000

_mesh = functools.partial(
    plsc.VectorSubcoreMesh,
    core_axis_name="c", subcore_axis_name="s", num_cores=NC, num_subcores=NS)


def _zero_acc(acc, zbuf, s, width):
    zeros16 = jnp.zeros((LN,), jnp.float32)

    def zrow(r, _):
        for u in range(width // LN):
            zbuf[r, pl.ds(u * LN, LN)] = zeros16
        return 0

    lax.fori_loop(0, ZROWS, zrow, 0)
    for j in range(ROWS_PER_SUB // ZROWS):
        pltpu.sync_copy(zbuf, acc.at[pl.ds(s * ROWS_PER_SUB + j * ZROWS, ZROWS)])


def _edge_range(s):
    base = s * (NCHUNK // NS) + jnp.minimum(s, NCHUNK % NS)
    n = jnp.where(s < NCHUNK % NS, NCHUNK // NS + 1, NCHUNK // NS)
    return base, n


EPS = E // NS
SEGC = 80
NB = EPS // SEGC
RING = 3
NMAIN = (NB - RING) // RING * RING


def _seg_body(x_hbm, gi_hbm, si_hbm, out_hbm,
              acc, srcall, dstb0, gbuf0, dstb1, gbuf1, dstb2, gbuf2,
              gsem0, gsem1, gsem2, ssem0, ssem1, ssem2,
              isem0, isem1, isem2):
    c = lax.axis_index("c")
    s = lax.axis_index("s")

    zeros16 = jnp.zeros((LN,), jnp.float32)

    def zrow(r, _):
        for u in range(HH // LN):
            gbuf0[r, pl.ds(u * LN, LN)] = zeros16
        return 0

    lax.fori_loop(0, SEGC, zrow, 0)
    r0 = s * ROWS_PER_SUB
    for j in range(ROWS_PER_SUB // SEGC):
        pltpu.sync_copy(gbuf0, acc.at[pl.ds(r0 + j * SEGC, SEGC)])
    rem = ROWS_PER_SUB % SEGC
    pltpu.sync_copy(gbuf0.at[pl.ds(0, rem)],
                    acc.at[pl.ds(r0 + ROWS_PER_SUB - rem, rem)])

    e0 = s * EPS
    pltpu.sync_copy(gi_hbm.at[pl.ds(e0, EPS)], srcall)

    def sfix(g, _):
        sl = pl.ds(g * LN, LN)
        srcall[sl] = srcall[sl] * 2 + c
        return 0

    lax.fori_loop(0, EPS // LN, sfix, 0)
    plsc.subcore_barrier()

    dstb = (dstb0, dstb1, dstb2)
    gbuf = (gbuf0, gbuf1, gbuf2)
    gsem = (gsem0, gsem1, gsem2)
    ssem = (ssem0, ssem1, ssem2)
    isem = (isem0, isem1, isem2)

    def idx_start(k, b):
        return pltpu.async_copy(si_hbm.at[pl.ds(e0 + k * SEGC, SEGC)],
                                dstb[b], isem[b])

    def idx_wait(k, b):
        pltpu.make_async_copy(si_hbm.at[pl.ds(e0 + k * SEGC, SEGC)],
                              dstb[b], isem[b]).wait()

    def gat_start(k, b):
        return pltpu.async_copy(
            x_hbm.at[srcall.at[pl.ds(k * SEGC, SEGC)]], gbuf[b], gsem[b])

    def gat_wait(k, b):
        pltpu.make_async_copy(
            x_hbm.at[srcall.at[pl.ds(k * SEGC, SEGC)]], gbuf[b],
            gsem[b]).wait()

    def sca_start(b):
        return pltpu.async_copy(gbuf[b], acc.at[dstb[b]], ssem[b], add=True)

    def sca_wait(b):
        pltpu.make_async_copy(gbuf[b], acc.at[dstb[b]], ssem[b]).wait()

    for b in range(RING):
        idx_start(b, b)
        gat_start(b, b)

    def ring_iter(k3, _):
        for j in range(RING):
            ok = RING * k3 + j
            idx_wait(ok, j)
            gat_wait(ok, j)
            sca_start(j)
            sca_wait(j)
            nk = ok + RING
            idx_start(nk, j)
            gat_start(nk, j)
        return 0

    lax.fori_loop(0, NMAIN // RING, ring_iter, 0)

    for t in range(NMAIN, NB):
        b = t % RING
        idx_wait(t, b)
        gat_wait(t, b)
        sca_start(b)
        sca_wait(b)
        nk = t + RING
        if nk < NB:
            idx_start(nk, b)
            gat_start(nk, b)

    plsc.subcore_barrier()

    @pl.when(s == 0)
    def _():
        pltpu.sync_copy(acc, out_hbm.at[c])


_seg = pl.kernel(
    _seg_body,
    out_type=jax.ShapeDtypeStruct((NC, N, HH), jnp.float32),
    mesh=_mesh(),
    scratch_types=[
        pltpu.VMEM_SHARED((N, HH), jnp.float32),
        pltpu.VMEM((EPS,), jnp.int32),
        pltpu.VMEM((SEGC,), jnp.int32),
        pltpu.VMEM((SEGC, HH), jnp.float32),
        pltpu.VMEM((SEGC,), jnp.int32),
        pltpu.VMEM((SEGC, HH), jnp.float32),
        pltpu.VMEM((SEGC,), jnp.int32),
        pltpu.VMEM((SEGC, HH), jnp.float32),
        pltpu.SemaphoreType.DMA,
        pltpu.SemaphoreType.DMA,
        pltpu.SemaphoreType.DMA,
        pltpu.SemaphoreType.DMA,
        pltpu.SemaphoreType.DMA,
        pltpu.SemaphoreType.DMA,
        pltpu.SemaphoreType.DMA,
        pltpu.SemaphoreType.DMA,
        pltpu.SemaphoreType.DMA,
    ],
)


def _cnt_body(dst_hbm, src_hbm, out_hbm, cacc, idxb, oneb, zbuf, sem):
    del sem
    c = lax.axis_index("c")
    s = lax.axis_index("s")

    _zero_acc(cacc, zbuf, s, HH)

    onehot = jnp.where(
        lax.broadcasted_iota(jnp.int32, (LN,), 0) == 0, 1.0, 0.0
    ).astype(jnp.float32)
    zeros16 = jnp.zeros((LN,), jnp.float32)

    def orow(r, _):
        oneb[r, pl.ds(0, LN)] = onehot
        for u in range(1, HH // LN):
            oneb[r, pl.ds(u * LN, LN)] = zeros16
        return 0

    lax.fori_loop(0, CHUNK, orow, 0)
    plsc.subcore_barrier()

    base_batch, n_batch = _edge_range(s)

    def edge_batch(k, _):
        off = (base_batch + k) * CHUNK

        @pl.when(c == 0)
        def _():
            pltpu.sync_copy(dst_hbm.at[pl.ds(off, CHUNK)], idxb)

        @pl.when(c == 1)
        def _():
            pltpu.sync_copy(src_hbm.at[pl.ds(off, CHUNK)], idxb)

        pltpu.sync_copy(oneb, cacc.at[idxb], add=True)
        return 0

    lax.fori_loop(0, n_batch, edge_batch, 0)
    plsc.subcore_barrier()

    @pl.when(s == 0)
    def _():
        pltpu.sync_copy(cacc, out_hbm.at[c])


_cnt_kernel = pl.kernel(
    _cnt_body,
    out_type=jax.ShapeDtypeStruct((NC, N, HH), jnp.float32),
    mesh=_mesh(),
    scratch_types=[
        pltpu.VMEM_SHARED((N, HH), jnp.float32),
        pltpu.VMEM((CHUNK,), jnp.int32),
        pltpu.VMEM((CHUNK, HH), jnp.float32),
        pltpu.VMEM((ZROWS, HH), jnp.float32),
        pltpu.SemaphoreType.DMA,
    ],
)


def _gather2_body(zp_hbm, zs_hbm, ia_hbm, ib_hbm, op_hbm, os_hbm,
                  iab, ibb, ga, gb, sem):
    c = lax.axis_index("c")
    s = lax.axis_index("s")
    wid = s * NC + c
    per_w = L // (NC * NS)
    for j in range(per_w // CHUNK):
        base = wid * per_w + j * CHUNK
        pltpu.sync_copy(ia_hbm.at[pl.ds(base, CHUNK)], iab)
        pltpu.sync_copy(ib_hbm.at[pl.ds(base, CHUNK)], ibb)
        pltpu.async_copy(zp_hbm.at[iab], ga, sem).wait()
        pltpu.async_copy(zs_hbm.at[ibb], gb, sem).wait()
        pltpu.sync_copy(ga, op_hbm.at[pl.ds(base, CHUNK)])
        pltpu.sync_copy(gb, os_hbm.at[pl.ds(base, CHUNK)])


_gather2 = pl.kernel(
    _gather2_body,
    out_type=(jax.ShapeDtypeStruct((L, H), jnp.float32),
              jax.ShapeDtypeStruct((L, H), jnp.float32)),
    mesh=_mesh(),
    scratch_types=[
        pltpu.VMEM((CHUNK,), jnp.int32),
        pltpu.VMEM((CHUNK,), jnp.int32),
        pltpu.VMEM((CHUNK, H), jnp.float32),
        pltpu.VMEM((CHUNK, H), jnp.float32),
        pltpu.SemaphoreType.DMA,
    ],
)


def _dot_body(a_ref, b_ref, o_ref):
    o_ref[...] = jnp.sum(a_ref[...] * b_ref[...], axis=1, keepdims=True)


def _rowdot(a, b):
    bm = 1024
    return pl.pallas_call(
        _dot_body,
        grid=(L // bm,),
        in_specs=[
            pl.BlockSpec((bm, H), lambda i: (i, 0)),
            pl.BlockSpec((bm, H), lambda i: (i, 0)),
        ],
        out_specs=pl.BlockSpec((bm, 1), lambda i: (i, 0)),
        out_shape=jax.ShapeDtypeStruct((L, 1), jnp.float32),
    )(a, b)


def _enc_body(x_ref, w_ref, b_ref, e_ref, o_ref):
    o_ref[...] = (jnp.dot(x_ref[...], w_ref[...],
                          preferred_element_type=jnp.float32)
                  + b_ref[...] + e_ref[...])


def _encoder(x, w, b, emb):
    M, K = x.shape
    return pl.pallas_call(
        _enc_body,
        grid=(M // BM,),
        in_specs=[
            pl.BlockSpec((BM, K), lambda i: (i, 0)),
            pl.BlockSpec((K, H), lambda i: (0, 0)),
            pl.BlockSpec((1, H), lambda i: (0, 0)),
            pl.BlockSpec((BM, H), lambda i: (i, 0)),
        ],
        out_specs=pl.BlockSpec((BM, H), lambda i: (i, 0)),
        out_shape=jax.ShapeDtypeStruct((M, H), jnp.float32),
    )(x, w, b.reshape(1, H), emb)


def _comb_body(relu, agg_a, cnt_a, xd_a, wn_a, ws_a, b_a,
               agg_b, cnt_b, xd_b, wn_b, ws_b, b_b, oa_ref, ob_ref):
    def one(agg_ref, cnt_ref, xd_ref, wn_ref, ws_ref, b_ref, o_ref):
        su = (jnp.dot(agg_ref[0], wn_ref[0],
                      preferred_element_type=jnp.float32)
              + jnp.dot(agg_ref[1], wn_ref[1],
                        preferred_element_type=jnp.float32))
        inv = 1.0 / jnp.maximum(cnt_ref[0][:, 0:1], 1.0)
        r = (su * inv
             + jnp.dot(xd_ref[...], ws_ref[...],
                       preferred_element_type=jnp.float32)
             + b_ref[...])
        o_ref[...] = jnp.maximum(r, 0.0) if relu else r

    one(agg_a, cnt_a, xd_a, wn_a, ws_a, b_a, oa_ref)
    one(agg_b, cnt_b, xd_b, wn_b, ws_b, b_b, ob_ref)


def _combine2(cnts, agg_a, xd_a, wn_a, ws_a, b_a,
              agg_b, xd_b, wn_b, ws_b, b_b, relu):
    set_a = [
        pl.BlockSpec((NC, BM, HH), lambda i: (0, i, 0)),
        pl.BlockSpec((1, BM, HH), lambda i: (0, i, 0)),
        pl.BlockSpec((BM, H), lambda i: (i, 0)),
        pl.BlockSpec((NC, HH, H), lambda i: (0, 0, 0)),
        pl.BlockSpec((H, H), lambda i: (0, 0)),
        pl.BlockSpec((1, H), lambda i: (0, 0)),
    ]
    set_b = list(set_a)
    set_b[1] = pl.BlockSpec((1, BM, HH), lambda i: (1, i, 0))
    return pl.pallas_call(
        functools.partial(_comb_body, relu),
        grid=(N // BM,),
        in_specs=set_a + set_b,
        out_specs=[pl.BlockSpec((BM, H), lambda i: (i, 0))] * 2,
        out_shape=[jax.ShapeDtypeStruct((N, H), jnp.float32)] * 2,
    )(agg_a, cnts, xd_a, wn_a.reshape(NC, HH, H), ws_a, b_a.reshape(1, H),
      agg_b, cnts, xd_b, wn_b.reshape(NC, HH, H), ws_b, b_b.reshape(1, H))


def kernel(pdrugs_x, seffect_x, pdrugs_node_id, seffect_node_id, edge_index,
           edge_label_index, edge_label, W_pd, b_pd, W_se, b_se, emb_pd,
           emb_se, l1_ps_Wn, l1_ps_Ws, l1_ps_b, l1_sp_Wn, l1_sp_Ws, l1_sp_b,
           l2_ps_Wn, l2_ps_Ws, l2_ps_b, l2_sp_Wn, l2_sp_Ws, l2_sp_b):
    x_pd = _encoder(pdrugs_x, W_pd, b_pd, emb_pd)
    x_se = _encoder(seffect_x, W_se, b_se, emb_se)
    src, dst = edge_index[0], edge_index[1]

    cnts = _cnt_kernel(dst, src)

    agg1_se = _seg(x_pd.reshape(2 * N, HH), src, dst)
    agg1_pd = _seg(x_se.reshape(2 * N, HH), dst, src)
    h_se, h_pd = _combine2(cnts,
                           agg1_se, x_se, l1_ps_Wn, l1_ps_Ws, l1_ps_b,
                           agg1_pd, x_pd, l1_sp_Wn, l1_sp_Ws, l1_sp_b, True)

    agg2_se = _seg(h_pd.reshape(2 * N, HH), src, dst)
    agg2_pd = _seg(h_se.reshape(2 * N, HH), dst, src)
    z_se, z_pd = _combine2(cnts,
                           agg2_se, h_se, l2_ps_Wn, l2_ps_Ws, l2_ps_b,
                           agg2_pd, h_pd, l2_sp_Wn, l2_sp_Ws, l2_sp_b, False)

    gp, gs = _gather2(z_pd, z_se, edge_label_index[0], edge_label_index[1])
    pred = _rowdot(gp, gs).reshape(L)
    return (pred, edge_label, edge_label_index)


# speedup vs baseline: 1.0997x; 1.0997x over previous
import functools

import jax
import jax.numpy as jnp
from jax import lax
from jax.experimental import pallas as pl
from jax.experimental.pallas import tpu as pltpu
from jax.experimental.pallas import tpu_sc as plsc

N = 10000
E = 160000
L = 8192
H = 256
HH = H // 2
NC, NS, LN = 2, 16, 16
CHUNK = 128
NCHUNK = E // CHUNK
ROWS_PER_SUB = N // NS
ZROWS = 25
BM = 1000

_mesh = functools.partial(
    plsc.VectorSubcoreMesh,
    core_axis_name="c", subcore_axis_name="s", num_cores=NC, num_subcores=NS)


def _zero_acc(acc, zbuf, s, width):
    zeros16 = jnp.zeros((LN,), jnp.float32)

    def zrow(r, _):
        for u in range(width // LN):
            zbuf[r, pl.ds(u * LN, LN)] = zeros16
        return 0

    lax.fori_loop(0, ZROWS, zrow, 0)
    for j in range(ROWS_PER_SUB // ZROWS):
        pltpu.sync_copy(zbuf, acc.at[pl.ds(s * ROWS_PER_SUB + j * ZROWS, ZROWS)])


def _edge_range(s):
    base = s * (NCHUNK // NS) + jnp.minimum(s, NCHUNK % NS)
    n = jnp.where(s < NCHUNK % NS, NCHUNK // NS + 1, NCHUNK // NS)
    return base, n


EPS = E // NS
SEGC = 80
NB = EPS // SEGC
RING = 3
NMAIN = (NB - RING) // RING * RING


def _seg_body(x_hbm, gi_hbm, si_hbm, out_hbm,
              acc, srcall, dstb0, gbuf0, dstb1, gbuf1, dstb2, gbuf2,
              gsem0, gsem1, gsem2, ssem0, ssem1, ssem2,
              isem0, isem1, isem2):
    c = lax.axis_index("c")
    s = lax.axis_index("s")

    zeros16 = jnp.zeros((LN,), jnp.float32)

    def zrow(r, _):
        for u in range(HH // LN):
            gbuf0[r, pl.ds(u * LN, LN)] = zeros16
        return 0

    lax.fori_loop(0, SEGC, zrow, 0)
    r0 = s * ROWS_PER_SUB
    for j in range(ROWS_PER_SUB // SEGC):
        pltpu.sync_copy(gbuf0, acc.at[pl.ds(r0 + j * SEGC, SEGC)])
    rem = ROWS_PER_SUB % SEGC
    pltpu.sync_copy(gbuf0.at[pl.ds(0, rem)],
                    acc.at[pl.ds(r0 + ROWS_PER_SUB - rem, rem)])

    e0 = s * EPS
    pltpu.sync_copy(gi_hbm.at[pl.ds(e0, EPS)], srcall)

    def sfix(g, _):
        sl = pl.ds(g * LN, LN)
        srcall[sl] = srcall[sl] * 2 + c
        return 0

    lax.fori_loop(0, EPS // LN, sfix, 0)
    plsc.subcore_barrier()

    dstb = (dstb0, dstb1, dstb2)
    gbuf = (gbuf0, gbuf1, gbuf2)
    gsem = (gsem0, gsem1, gsem2)
    ssem = (ssem0, ssem1, ssem2)
    isem = (isem0, isem1, isem2)

    def idx_start(k, b):
        return pltpu.async_copy(si_hbm.at[pl.ds(e0 + k * SEGC, SEGC)],
                                dstb[b], isem[b])

    def idx_wait(k, b):
        pltpu.make_async_copy(si_hbm.at[pl.ds(e0 + k * SEGC, SEGC)],
                              dstb[b], isem[b]).wait()

    def gat_start(k, b):
        return pltpu.async_copy(
            x_hbm.at[srcall.at[pl.ds(k * SEGC, SEGC)]], gbuf[b], gsem[b])

    def gat_wait(k, b):
        pltpu.make_async_copy(
            x_hbm.at[srcall.at[pl.ds(k * SEGC, SEGC)]], gbuf[b],
            gsem[b]).wait()

    def sca_start(b):
        return pltpu.async_copy(gbuf[b], acc.at[dstb[b]], ssem[b], add=True)

    def sca_wait(b):
        pltpu.make_async_copy(gbuf[b], acc.at[dstb[b]], ssem[b]).wait()

    for b in range(RING):
        idx_start(b, b)
        gat_start(b, b)

    def ring_iter(k3, _):
        for j in range(RING):
            ok = RING * k3 + j
            idx_wait(ok, j)
            gat_wait(ok, j)
            sca_start(j)
            sca_wait(j)
            nk = ok + RING
            idx_start(nk, j)
            gat_start(nk, j)
        return 0

    lax.fori_loop(0, NMAIN // RING, ring_iter, 0)

    for t in range(NMAIN, NB):
        b = t % RING
        idx_wait(t, b)
        gat_wait(t, b)
        sca_start(b)
        sca_wait(b)
        nk = t + RING
        if nk < NB:
            idx_start(nk, b)
            gat_start(nk, b)

    plsc.subcore_barrier()

    @pl.when(s == 0)
    def _():
        pltpu.sync_copy(acc, out_hbm.at[c])


_seg = pl.kernel(
    _seg_body,
    out_type=jax.ShapeDtypeStruct((NC, N, HH), jnp.float32),
    mesh=_mesh(),
    scratch_types=[
        pltpu.VMEM_SHARED((N, HH), jnp.float32),
        pltpu.VMEM((EPS,), jnp.int32),
        pltpu.VMEM((SEGC,), jnp.int32),
        pltpu.VMEM((SEGC, HH), jnp.float32),
        pltpu.VMEM((SEGC,), jnp.int32),
        pltpu.VMEM((SEGC, HH), jnp.float32),
        pltpu.VMEM((SEGC,), jnp.int32),
        pltpu.VMEM((SEGC, HH), jnp.float32),
        pltpu.SemaphoreType.DMA,
        pltpu.SemaphoreType.DMA,
        pltpu.SemaphoreType.DMA,
        pltpu.SemaphoreType.DMA,
        pltpu.SemaphoreType.DMA,
        pltpu.SemaphoreType.DMA,
        pltpu.SemaphoreType.DMA,
        pltpu.SemaphoreType.DMA,
        pltpu.SemaphoreType.DMA,
    ],
)


def _cnt_body(dst_hbm, src_hbm, out_hbm, cacc, idxb, oneb, zbuf, sem):
    del sem
    c = lax.axis_index("c")
    s = lax.axis_index("s")

    _zero_acc(cacc, zbuf, s, HH)

    onehot = jnp.where(
        lax.broadcasted_iota(jnp.int32, (LN,), 0) == 0, 1.0, 0.0
    ).astype(jnp.float32)
    zeros16 = jnp.zeros((LN,), jnp.float32)

    def orow(r, _):
        oneb[r, pl.ds(0, LN)] = onehot
        for u in range(1, HH // LN):
            oneb[r, pl.ds(u * LN, LN)] = zeros16
        return 0

    lax.fori_loop(0, CHUNK, orow, 0)
    plsc.subcore_barrier()

    base_batch, n_batch = _edge_range(s)

    def edge_batch(k, _):
        off = (base_batch + k) * CHUNK

        @pl.when(c == 0)
        def _():
            pltpu.sync_copy(dst_hbm.at[pl.ds(off, CHUNK)], idxb)

        @pl.when(c == 1)
        def _():
            pltpu.sync_copy(src_hbm.at[pl.ds(off, CHUNK)], idxb)

        pltpu.sync_copy(oneb, cacc.at[idxb], add=True)
        return 0

    lax.fori_loop(0, n_batch, edge_batch, 0)
    plsc.subcore_barrier()

    @pl.when(s == 0)
    def _():
        pltpu.sync_copy(cacc, out_hbm.at[c])


_cnt_kernel = pl.kernel(
    _cnt_body,
    out_type=jax.ShapeDtypeStruct((NC, N, HH), jnp.float32),
    mesh=_mesh(),
    scratch_types=[
        pltpu.VMEM_SHARED((N, HH), jnp.float32),
        pltpu.VMEM((CHUNK,), jnp.int32),
        pltpu.VMEM((CHUNK, HH), jnp.float32),
        pltpu.VMEM((ZROWS, HH), jnp.float32),
        pltpu.SemaphoreType.DMA,
    ],
)


def _gather2_body(zp_hbm, zs_hbm, ia_hbm, ib_hbm, op_hbm, os_hbm,
                  iab, ibb, ga, gb, sem):
    c = lax.axis_index("c")
    s = lax.axis_index("s")
    wid = s * NC + c
    per_w = L // (NC * NS)
    for j in range(per_w // CHUNK):
        base = wid * per_w + j * CHUNK
        pltpu.sync_copy(ia_hbm.at[pl.ds(base, CHUNK)], iab)
        pltpu.sync_copy(ib_hbm.at[pl.ds(base, CHUNK)], ibb)
        pltpu.async_copy(zp_hbm.at[iab], ga, sem).wait()
        pltpu.async_copy(zs_hbm.at[ibb], gb, sem).wait()
        pltpu.sync_copy(ga, op_hbm.at[pl.ds(base, CHUNK)])
        pltpu.sync_copy(gb, os_hbm.at[pl.ds(base, CHUNK)])


_gather2 = pl.kernel(
    _gather2_body,
    out_type=(jax.ShapeDtypeStruct((L, H), jnp.float32),
              jax.ShapeDtypeStruct((L, H), jnp.float32)),
    mesh=_mesh(),
    scratch_types=[
        pltpu.VMEM((CHUNK,), jnp.int32),
        pltpu.VMEM((CHUNK,), jnp.int32),
        pltpu.VMEM((CHUNK, H), jnp.float32),
        pltpu.VMEM((CHUNK, H), jnp.float32),
        pltpu.SemaphoreType.DMA,
    ],
)


def _dot_body(a_ref, b_ref, o_ref):
    o_ref[...] = jnp.sum(a_ref[...] * b_ref[...], axis=1, keepdims=True)


def _rowdot(a, b):
    bm = 1024
    return pl.pallas_call(
        _dot_body,
        grid=(L // bm,),
        in_specs=[
            pl.BlockSpec((bm, H), lambda i: (i, 0)),
            pl.BlockSpec((bm, H), lambda i: (i, 0)),
        ],
        out_specs=pl.BlockSpec((bm, 1), lambda i: (i, 0)),
        out_shape=jax.ShapeDtypeStruct((L, 1), jnp.float32),
    )(a, b)


def _enc_body(x_ref, w_ref, b_ref, e_ref, o_ref):
    o_ref[...] = (jnp.dot(x_ref[...], w_ref[...],
                          preferred_element_type=jnp.float32)
                  + b_ref[...] + e_ref[...])


def _encoder(x, w, b, emb):
    M, K = x.shape
    return pl.pallas_call(
        _enc_body,
        grid=(M // BM,),
        in_specs=[
            pl.BlockSpec((BM, K), lambda i: (i, 0)),
            pl.BlockSpec((K, H), lambda i: (0, 0)),
            pl.BlockSpec((1, H), lambda i: (0, 0)),
            pl.BlockSpec((BM, H), lambda i: (i, 0)),
        ],
        out_specs=pl.BlockSpec((BM, H), lambda i: (i, 0)),
        out_shape=jax.ShapeDtypeStruct((M, H), jnp.float32),
    )(x, w, b.reshape(1, H), emb)


def _comb_body(relu, agg_ref, cnt_ref, xd_ref, wn_ref, ws_ref, b_ref, o_ref):
    su = (jnp.dot(agg_ref[0], wn_ref[0], preferred_element_type=jnp.float32)
          + jnp.dot(agg_ref[1], wn_ref[1], preferred_element_type=jnp.float32))
    inv = 1.0 / jnp.maximum(cnt_ref[0][:, 0:1], 1.0)
    r = (su * inv
         + jnp.dot(xd_ref[...], ws_ref[...], preferred_element_type=jnp.float32)
         + b_ref[...])
    o_ref[...] = jnp.maximum(r, 0.0) if relu else r


def _combine(agg, cnts, d, xd, wn, ws, b, relu):
    return pl.pallas_call(
        functools.partial(_comb_body, relu),
        grid=(N // BM,),
        in_specs=[
            pl.BlockSpec((NC, BM, HH), lambda i: (0, i, 0)),
            pl.BlockSpec((1, BM, HH), lambda i, d=d: (d, i, 0)),
            pl.BlockSpec((BM, H), lambda i: (i, 0)),
            pl.BlockSpec((NC, HH, H), lambda i: (0, 0, 0)),
            pl.BlockSpec((H, H), lambda i: (0, 0)),
            pl.BlockSpec((1, H), lambda i: (0, 0)),
        ],
        out_specs=pl.BlockSpec((BM, H), lambda i: (i, 0)),
        out_shape=jax.ShapeDtypeStruct((N, H), jnp.float32),
    )(agg, cnts, xd, wn.reshape(NC, HH, H), ws, b.reshape(1, H))


def kernel(pdrugs_x, seffect_x, pdrugs_node_id, seffect_node_id, edge_index,
           edge_label_index, edge_label, W_pd, b_pd, W_se, b_se, emb_pd,
           emb_se, l1_ps_Wn, l1_ps_Ws, l1_ps_b, l1_sp_Wn, l1_sp_Ws, l1_sp_b,
           l2_ps_Wn, l2_ps_Ws, l2_ps_b, l2_sp_Wn, l2_sp_Ws, l2_sp_b):
    x_pd = _encoder(pdrugs_x, W_pd, b_pd, emb_pd)
    x_se = _encoder(seffect_x, W_se, b_se, emb_se)
    src, dst = edge_index[0], edge_index[1]

    cnts = _cnt_kernel(dst, src)

    agg1_se = _seg(x_pd.reshape(2 * N, HH), src, dst)
    agg1_pd = _seg(x_se.reshape(2 * N, HH), dst, src)
    h_se = _combine(agg1_se, cnts, 0, x_se, l1_ps_Wn, l1_ps_Ws, l1_ps_b, True)
    h_pd = _combine(agg1_pd, cnts, 1, x_pd, l1_sp_Wn, l1_sp_Ws, l1_sp_b, True)

    agg2_se = _seg(h_pd.reshape(2 * N, HH), src, dst)
    agg2_pd = _seg(h_se.reshape(2 * N, HH), dst, src)
    z_se = _combine(agg2_se, cnts, 0, h_se, l2_ps_Wn, l2_ps_Ws, l2_ps_b, False)
    z_pd = _combine(agg2_pd, cnts, 1, h_pd, l2_sp_Wn, l2_sp_Ws, l2_sp_b, False)

    gp, gs = _gather2(z_pd, z_se, edge_label_index[0], edge_label_index[1])
    pred = _rowdot(gp, gs).reshape(L)
    return (pred, edge_label, edge_label_index)


# speedup vs baseline: 1.1214x; 1.0198x over previous
import functools

import jax
import jax.numpy as jnp
from jax import lax
from jax.experimental import pallas as pl
from jax.experimental.pallas import tpu as pltpu
from jax.experimental.pallas import tpu_sc as plsc

N = 10000
E = 160000
L = 8192
H = 256
HH = H // 2
NC, NS, LN = 2, 16, 16
CHUNK = 128
NCHUNK = E // CHUNK
ROWS_PER_SUB = N // NS
ZROWS = 25
BM = 1000

_mesh = functools.partial(
    plsc.VectorSubcoreMesh,
    core_axis_name="c", subcore_axis_name="s", num_cores=NC, num_subcores=NS)


def _zero_acc(acc, zbuf, s, width):
    zeros16 = jnp.zeros((LN,), jnp.float32)

    def zrow(r, _):
        for u in range(width // LN):
            zbuf[r, pl.ds(u * LN, LN)] = zeros16
        return 0

    lax.fori_loop(0, ZROWS, zrow, 0)
    for j in range(ROWS_PER_SUB // ZROWS):
        pltpu.sync_copy(zbuf, acc.at[pl.ds(s * ROWS_PER_SUB + j * ZROWS, ZROWS)])


def _edge_range(s):
    base = s * (NCHUNK // NS) + jnp.minimum(s, NCHUNK % NS)
    n = jnp.where(s < NCHUNK % NS, NCHUNK // NS + 1, NCHUNK // NS)
    return base, n


EPS = E // NS
SEGC = 80
NB = EPS // SEGC
RING = 3
NMAIN = (NB - RING) // RING * RING


def _seg_body(x_hbm, gi_hbm, si_hbm, out_hbm,
              acc, srcall, dstb0, gbuf0, dstb1, gbuf1, dstb2, gbuf2,
              gsem0, gsem1, gsem2, ssem0, ssem1, ssem2,
              isem0, isem1, isem2):
    c = lax.axis_index("c")
    s = lax.axis_index("s")

    zeros16 = jnp.zeros((LN,), jnp.float32)

    def zrow(r, _):
        for u in range(HH // LN):
            gbuf0[r, pl.ds(u * LN, LN)] = zeros16
        return 0

    lax.fori_loop(0, SEGC, zrow, 0)
    r0 = s * ROWS_PER_SUB
    rem = ROWS_PER_SUB % SEGC
    for j in range(ROWS_PER_SUB // SEGC):
        pltpu.async_copy(gbuf0, acc.at[pl.ds(r0 + j * SEGC, SEGC)], isem0)
    pltpu.async_copy(gbuf0.at[pl.ds(0, rem)],
                     acc.at[pl.ds(r0 + ROWS_PER_SUB - rem, rem)], isem0)

    e0 = s * EPS
    pltpu.sync_copy(gi_hbm.at[pl.ds(e0, EPS)], srcall)

    def sfix(g, _):
        sl = pl.ds(g * LN, LN)
        srcall[sl] = srcall[sl] * 2 + c
        return 0

    lax.fori_loop(0, EPS // LN, sfix, 0)
    for j in range(ROWS_PER_SUB // SEGC):
        pltpu.make_async_copy(
            gbuf0, acc.at[pl.ds(r0 + j * SEGC, SEGC)], isem0).wait()
    pltpu.make_async_copy(
        gbuf0.at[pl.ds(0, rem)],
        acc.at[pl.ds(r0 + ROWS_PER_SUB - rem, rem)], isem0).wait()
    plsc.subcore_barrier()

    dstb = (dstb0, dstb1, dstb2)
    gbuf = (gbuf0, gbuf1, gbuf2)
    gsem = (gsem0, gsem1, gsem2)
    ssem = (ssem0, ssem1, ssem2)
    isem = (isem0, isem1, isem2)

    def idx_start(k, b):
        return pltpu.async_copy(si_hbm.at[pl.ds(e0 + k * SEGC, SEGC)],
                                dstb[b], isem[b])

    def idx_wait(k, b):
        pltpu.make_async_copy(si_hbm.at[pl.ds(e0 + k * SEGC, SEGC)],
                              dstb[b], isem[b]).wait()

    def gat_start(k, b):
        return pltpu.async_copy(
            x_hbm.at[srcall.at[pl.ds(k * SEGC, SEGC)]], gbuf[b], gsem[b])

    def gat_wait(k, b):
        pltpu.make_async_copy(
            x_hbm.at[srcall.at[pl.ds(k * SEGC, SEGC)]], gbuf[b],
            gsem[b]).wait()

    def sca_start(b):
        return pltpu.async_copy(gbuf[b], acc.at[dstb[b]], ssem[b], add=True)

    def sca_wait(b):
        pltpu.make_async_copy(gbuf[b], acc.at[dstb[b]], ssem[b]).wait()

    for b in range(RING):
        idx_start(b, b)
        gat_start(b, b)

    def ring_iter(k3, _):
        for j in range(RING):
            ok = RING * k3 + j
            idx_wait(ok, j)
            gat_wait(ok, j)
            sca_start(j)
            sca_wait(j)
            nk = ok + RING
            idx_start(nk, j)
            gat_start(nk, j)
        return 0

    lax.fori_loop(0, NMAIN // RING, ring_iter, 0)

    for t in range(NMAIN, NB):
        b = t % RING
        idx_wait(t, b)
        gat_wait(t, b)
        sca_start(b)
        sca_wait(b)
        nk = t + RING
        if nk < NB:
            idx_start(nk, b)
            gat_start(nk, b)

    plsc.subcore_barrier()

    @pl.when(s == 0)
    def _():
        pltpu.sync_copy(acc, out_hbm.at[c])


_seg = pl.kernel(
    _seg_body,
    out_type=jax.ShapeDtypeStruct((NC, N, HH), jnp.float32),
    mesh=_mesh(),
    scratch_types=[
        pltpu.VMEM_SHARED((N, HH), jnp.float32),
        pltpu.VMEM((EPS,), jnp.int32),
        pltpu.VMEM((SEGC,), jnp.int32),
        pltpu.VMEM((SEGC, HH), jnp.float32),
        pltpu.VMEM((SEGC,), jnp.int32),
        pltpu.VMEM((SEGC, HH), jnp.float32),
        pltpu.VMEM((SEGC,), jnp.int32),
        pltpu.VMEM((SEGC, HH), jnp.float32),
        pltpu.SemaphoreType.DMA,
        pltpu.SemaphoreType.DMA,
        pltpu.SemaphoreType.DMA,
        pltpu.SemaphoreType.DMA,
        pltpu.SemaphoreType.DMA,
        pltpu.SemaphoreType.DMA,
        pltpu.SemaphoreType.DMA,
        pltpu.SemaphoreType.DMA,
        pltpu.SemaphoreType.DMA,
    ],
)


def _cnt_body(dst_hbm, src_hbm, out_hbm, cacc, idxb, oneb, zbuf, sem):
    c = lax.axis_index("c")
    s = lax.axis_index("s")

    zeros16z = jnp.zeros((LN,), jnp.float32)

    def zrowz(r, _):
        for u in range(HH // LN):
            zbuf[r, pl.ds(u * LN, LN)] = zeros16z
        return 0

    lax.fori_loop(0, ZROWS, zrowz, 0)
    zslices = [pl.ds(s * ROWS_PER_SUB + j * ZROWS, ZROWS)
               for j in range(ROWS_PER_SUB // ZROWS)]
    for sl in zslices:
        pltpu.async_copy(zbuf, cacc.at[sl], sem)

    onehot = jnp.where(
        lax.broadcasted_iota(jnp.int32, (LN,), 0) == 0, 1.0, 0.0
    ).astype(jnp.float32)
    zeros16 = jnp.zeros((LN,), jnp.float32)

    def orow(r, _):
        oneb[r, pl.ds(0, LN)] = onehot
        for u in range(1, HH // LN):
            oneb[r, pl.ds(u * LN, LN)] = zeros16
        return 0

    lax.fori_loop(0, CHUNK, orow, 0)
    for sl in zslices:
        pltpu.make_async_copy(zbuf, cacc.at[sl], sem).wait()
    plsc.subcore_barrier()

    base_batch, n_batch = _edge_range(s)

    def edge_batch(k, _):
        off = (base_batch + k) * CHUNK

        @pl.when(c == 0)
        def _():
            pltpu.sync_copy(dst_hbm.at[pl.ds(off, CHUNK)], idxb)

        @pl.when(c == 1)
        def _():
            pltpu.sync_copy(src_hbm.at[pl.ds(off, CHUNK)], idxb)

        pltpu.sync_copy(oneb, cacc.at[idxb], add=True)
        return 0

    lax.fori_loop(0, n_batch, edge_batch, 0)
    plsc.subcore_barrier()

    @pl.when(s == 0)
    def _():
        pltpu.sync_copy(cacc, out_hbm.at[c])


_cnt_kernel = pl.kernel(
    _cnt_body,
    out_type=jax.ShapeDtypeStruct((NC, N, HH), jnp.float32),
    mesh=_mesh(),
    scratch_types=[
        pltpu.VMEM_SHARED((N, HH), jnp.float32),
        pltpu.VMEM((CHUNK,), jnp.int32),
        pltpu.VMEM((CHUNK, HH), jnp.float32),
        pltpu.VMEM((ZROWS, HH), jnp.float32),
        pltpu.SemaphoreType.DMA,
    ],
)


def _gather2_body(zp_hbm, zs_hbm, ia_hbm, ib_hbm, op_hbm, os_hbm,
                  iab, ibb, ga, gb, sem):
    c = lax.axis_index("c")
    s = lax.axis_index("s")
    wid = s * NC + c
    per_w = L // (NC * NS)
    for j in range(per_w // CHUNK):
        base = wid * per_w + j * CHUNK
        pltpu.sync_copy(ia_hbm.at[pl.ds(base, CHUNK)], iab)
        pltpu.sync_copy(ib_hbm.at[pl.ds(base, CHUNK)], ibb)
        pltpu.async_copy(zp_hbm.at[iab], ga, sem).wait()
        pltpu.async_copy(zs_hbm.at[ibb], gb, sem).wait()
        pltpu.sync_copy(ga, op_hbm.at[pl.ds(base, CHUNK)])
        pltpu.sync_copy(gb, os_hbm.at[pl.ds(base, CHUNK)])


_gather2 = pl.kernel(
    _gather2_body,
    out_type=(jax.ShapeDtypeStruct((L, H), jnp.float32),
              jax.ShapeDtypeStruct((L, H), jnp.float32)),
    mesh=_mesh(),
    scratch_types=[
        pltpu.VMEM((CHUNK,), jnp.int32),
        pltpu.VMEM((CHUNK,), jnp.int32),
        pltpu.VMEM((CHUNK, H), jnp.float32),
        pltpu.VMEM((CHUNK, H), jnp.float32),
        pltpu.SemaphoreType.DMA,
    ],
)


def _dot_body(a_ref, b_ref, o_ref):
    o_ref[...] = jnp.sum(a_ref[...] * b_ref[...], axis=1, keepdims=True)


def _rowdot(a, b):
    bm = 1024
    return pl.pallas_call(
        _dot_body,
        grid=(L // bm,),
        in_specs=[
            pl.BlockSpec((bm, H), lambda i: (i, 0)),
            pl.BlockSpec((bm, H), lambda i: (i, 0)),
        ],
        out_specs=pl.BlockSpec((bm, 1), lambda i: (i, 0)),
        out_shape=jax.ShapeDtypeStruct((L, 1), jnp.float32),
    )(a, b)


def _enc_body(x_ref, w_ref, b_ref, e_ref, o_ref):
    o_ref[...] = (jnp.dot(x_ref[...], w_ref[...],
                          preferred_element_type=jnp.float32)
                  + b_ref[...] + e_ref[...])


def _encoder(x, w, b, emb):
    M, K = x.shape
    return pl.pallas_call(
        _enc_body,
        grid=(M // BM,),
        in_specs=[
            pl.BlockSpec((BM, K), lambda i: (i, 0)),
            pl.BlockSpec((K, H), lambda i: (0, 0)),
            pl.BlockSpec((1, H), lambda i: (0, 0)),
            pl.BlockSpec((BM, H), lambda i: (i, 0)),
        ],
        out_specs=pl.BlockSpec((BM, H), lambda i: (i, 0)),
        out_shape=jax.ShapeDtypeStruct((M, H), jnp.float32),
    )(x, w, b.reshape(1, H), emb)


def _comb_body(relu, agg_ref, cnt_ref, xd_ref, wn_ref, ws_ref, b_ref, o_ref):
    su = (jnp.dot(agg_ref[0], wn_ref[0], preferred_element_type=jnp.float32)
          + jnp.dot(agg_ref[1], wn_ref[1], preferred_element_type=jnp.float32))
    inv = 1.0 / jnp.maximum(cnt_ref[0][:, 0:1], 1.0)
    r = (su * inv
         + jnp.dot(xd_ref[...], ws_ref[...], preferred_element_type=jnp.float32)
         + b_ref[...])
    o_ref[...] = jnp.maximum(r, 0.0) if relu else r


def _combine(agg, cnts, d, xd, wn, ws, b, relu):
    return pl.pallas_call(
        functools.partial(_comb_body, relu),
        grid=(N // BM,),
        in_specs=[
            pl.BlockSpec((NC, BM, HH), lambda i: (0, i, 0)),
            pl.BlockSpec((1, BM, HH), lambda i, d=d: (d, i, 0)),
            pl.BlockSpec((BM, H), lambda i: (i, 0)),
            pl.BlockSpec((NC, HH, H), lambda i: (0, 0, 0)),
            pl.BlockSpec((H, H), lambda i: (0, 0)),
            pl.BlockSpec((1, H), lambda i: (0, 0)),
        ],
        out_specs=pl.BlockSpec((BM, H), lambda i: (i, 0)),
        out_shape=jax.ShapeDtypeStruct((N, H), jnp.float32),
    )(agg, cnts, xd, wn.reshape(NC, HH, H), ws, b.reshape(1, H))


def kernel(pdrugs_x, seffect_x, pdrugs_node_id, seffect_node_id, edge_index,
           edge_label_index, edge_label, W_pd, b_pd, W_se, b_se, emb_pd,
           emb_se, l1_ps_Wn, l1_ps_Ws, l1_ps_b, l1_sp_Wn, l1_sp_Ws, l1_sp_b,
           l2_ps_Wn, l2_ps_Ws, l2_ps_b, l2_sp_Wn, l2_sp_Ws, l2_sp_b):
    x_pd = _encoder(pdrugs_x, W_pd, b_pd, emb_pd)
    x_se = _encoder(seffect_x, W_se, b_se, emb_se)
    src, dst = edge_index[0], edge_index[1]

    cnts = _cnt_kernel(dst, src)

    agg1_se = _seg(x_pd.reshape(2 * N, HH), src, dst)
    agg1_pd = _seg(x_se.reshape(2 * N, HH), dst, src)
    h_se = _combine(agg1_se, cnts, 0, x_se, l1_ps_Wn, l1_ps_Ws, l1_ps_b, True)
    h_pd = _combine(agg1_pd, cnts, 1, x_pd, l1_sp_Wn, l1_sp_Ws, l1_sp_b, True)

    agg2_se = _seg(h_pd.reshape(2 * N, HH), src, dst)
    agg2_pd = _seg(h_se.reshape(2 * N, HH), dst, src)
    z_se = _combine(agg2_se, cnts, 0, h_se, l2_ps_Wn, l2_ps_Ws, l2_ps_b, False)
    z_pd = _combine(agg2_pd, cnts, 1, h_pd, l2_sp_Wn, l2_sp_Ws, l2_sp_b, False)

    gp, gs = _gather2(z_pd, z_se, edge_label_index[0], edge_label_index[1])
    pred = _rowdot(gp, gs).reshape(L)
    return (pred, edge_label, edge_label_index)
